# baseline (speedup 1.0000x reference)
import functools

import jax
import jax.numpy as jnp
from jax import lax
from jax.experimental import pallas as pl
from jax.experimental.pallas import tpu as pltpu
from jax.experimental.pallas import tpu_sc as plsc

VOCAB = 1000000
EMBED_DIM = 16
PADDING_IDX = 0
BATCH = 4096
SEQ_LEN = 200
TOTAL = BATCH * SEQ_LEN

_INFO = plsc.get_sparse_core_info()
NC = _INFO.num_cores
NS = _INFO.num_subcores
NW = NC * NS
PER_W = TOTAL // NW
CHUNK = 2560
N_CHUNKS = PER_W // CHUNK

_T1_VB = 65536
_T1_GRID = (VOCAB + _T1_VB - 1) // _T1_VB
_PERM_ROWS = _T1_GRID * _T1_VB // 8
_PERM_VOCAB = _PERM_ROWS * 128 // EMBED_DIM


_SCH = 8
_SC_CHUNK = _SCH * 128
_SC_NCH = SEQ_LEN // _SCH


def _gather_body(tokens_hbm, table_hbm, out_hbm, idx_v, idx2_v,
                 rows_v, tbuf_v, gsem0, gsem1, wsem0, wsem1):
    wid = lax.axis_index("s") * NC + lax.axis_index("c")
    base = wid * PER_W
    gsems = (gsem0, gsem1)
    wsems = (wsem0, wsem1)

    pltpu.sync_copy(tokens_hbm.at[pl.ds(base, PER_W)], idx_v)

    iota = lax.iota(jnp.int32, 16)
    step200 = iota * 200

    def xform(j, _):
        s = j // 8
        b0 = (j % 8) * 16
        src = step200 + (b0 * 200 + s)
        v = plsc.load_gather(idx_v, [src])
        rho = (v & -1024) + ((v & 127) << 3) + ((v >> 7) & 7)
        idx2_v[pl.ds(j * 16, 16)] = rho
        return 0

    lax.fori_loop(0, PER_W // 16, xform, 0, unroll=4)

    def start_gather(ch, b):
        return pltpu.async_copy(
            table_hbm.at[idx2_v.at[pl.ds(ch * _SC_CHUNK, _SC_CHUNK)]],
            rows_v.at[b], gsems[b])

    def start_write(ch, b):
        return pltpu.async_copy(
            tbuf_v.at[b],
            out_hbm.at[pl.ds(ch * _SCH * EMBED_DIM, _SCH * EMBED_DIM), wid, :],
            wsems[b])

    def transpose_chunk(b):
        rows_b = rows_v.at[b]
        tbuf_b = tbuf_v.at[b]

        def tstep(t, _):
            sl = t >> 7
            d = (t >> 3) & 15
            q = t & 7
            row_idx = iota + (sl * 128 + q * 16)
            col_idx = jnp.full((16,), d, jnp.int32)
            vec = plsc.load_gather(rows_b, [row_idx, col_idx])
            tbuf_b[sl * 16 + d, pl.ds(q * 16, 16)] = vec
            return 0

        lax.fori_loop(0, _SCH * EMBED_DIM * 8, tstep, 0, unroll=4)

    g = [None, None]
    w = [None, None]
    g[0] = start_gather(0, 0)
    g[1] = start_gather(1, 1)
    for ch in range(_SC_NCH):
        b = ch & 1
        g[b].wait()
        if w[b] is not None:
            w[b].wait()
        transpose_chunk(b)
        w[b] = start_write(ch, b)
        if ch + 2 < _SC_NCH:
            g[b] = start_gather(ch + 2, b)
    w[0].wait()
    w[1].wait()


_gather = functools.partial(
    pl.kernel,
    out_type=jax.ShapeDtypeStruct((SEQ_LEN * EMBED_DIM, NW, 128),
                                  jnp.float32),
    mesh=plsc.VectorSubcoreMesh(core_axis_name="c", subcore_axis_name="s"),
    scratch_types=[
        pltpu.VMEM((PER_W,), jnp.int32),
        pltpu.VMEM((PER_W,), jnp.int32),
        pltpu.VMEM((2, _SC_CHUNK, EMBED_DIM), jnp.float32),
        pltpu.VMEM((2, _SCH * EMBED_DIM, 128), jnp.float32),
        pltpu.SemaphoreType.DMA,
        pltpu.SemaphoreType.DMA,
        pltpu.SemaphoreType.DMA,
        pltpu.SemaphoreType.DMA,
    ],
    compiler_params=pltpu.CompilerParams(use_tc_tiling_on_sc=False,
                                         needs_layout_passes=False),
)(_gather_body)


def _t1_body(tt_ref, out_ref):
    for k in range(_T1_VB // 1024):
        base = 1024 * k
        m = jnp.concatenate(
            [tt_ref[:, base + 128 * c:base + 128 * (c + 1)]
             for c in range(8)], axis=0)
        out_ref[128 * k:128 * (k + 1), :] = m.T


_table_to_rowmajor = pl.pallas_call(
    _t1_body,
    grid=(_T1_GRID,),
    in_specs=[pl.BlockSpec((16, _T1_VB), lambda i: (0, i))],
    out_specs=pl.BlockSpec((_T1_VB // 8, 128), lambda i: (i, 0)),
    out_shape=jax.ShapeDtypeStruct((_PERM_ROWS, 128), jnp.float32),
)

_T2_BB = 256


def _t2_body(rows_ref, out_ref):
    x = rows_ref[...]
    out_ref[...] = x.T.reshape(SEQ_LEN, EMBED_DIM, _T2_BB)


_rows_to_features = pl.pallas_call(
    _t2_body,
    grid=(BATCH // _T2_BB,),
    in_specs=[pl.BlockSpec((_T2_BB, SEQ_LEN * EMBED_DIM), lambda i: (i, 0))],
    out_specs=pl.BlockSpec((SEQ_LEN, EMBED_DIM, _T2_BB), lambda i: (0, 0, i)),
    out_shape=jax.ShapeDtypeStruct((SEQ_LEN, EMBED_DIM, BATCH), jnp.float32),
)


def _mask_body(tokens_ref, pad_ref, seq_ref):
    pad_ref[...] = tokens_ref[...] == PADDING_IDX
    row = lax.broadcasted_iota(jnp.int32, (SEQ_LEN, SEQ_LEN), 0)
    col = lax.broadcasted_iota(jnp.int32, (SEQ_LEN, SEQ_LEN), 1)
    seq_ref[...] = col > row


_masks = pl.pallas_call(
    _mask_body,
    out_shape=(
        jax.ShapeDtypeStruct((BATCH, SEQ_LEN), jnp.bool_),
        jax.ShapeDtypeStruct((SEQ_LEN, SEQ_LEN), jnp.bool_),
    ),
)


def kernel(tokens, table):
    flat = tokens.reshape(TOTAL)
    table_lin = (_table_to_rowmajor(table.T)
                 .reshape(_PERM_ROWS * 128)
                 .reshape(_PERM_VOCAB, EMBED_DIM))
    out3 = _gather(flat, table_lin)
    feat_t = (out3.reshape(SEQ_LEN * EMBED_DIM * NW * 128)
                  .reshape(SEQ_LEN, EMBED_DIM, BATCH))
    features = feat_t.transpose(2, 0, 1)
    pad, seqm = _masks(tokens)
    return (features, (pad, seqm))


# speedup vs baseline: 1.5965x; 1.5965x over previous
import functools

import jax
import jax.numpy as jnp
from jax import lax
from jax.experimental import pallas as pl
from jax.experimental.pallas import tpu as pltpu
from jax.experimental.pallas import tpu_sc as plsc

VOCAB = 1000000
EMBED_DIM = 16
PADDING_IDX = 0
BATCH = 4096
SEQ_LEN = 200
TOTAL = BATCH * SEQ_LEN

_INFO = plsc.get_sparse_core_info()
NC = _INFO.num_cores
NS = _INFO.num_subcores
NW = NC * NS
PER_W = TOTAL // NW
CHUNK = 2560
N_CHUNKS = PER_W // CHUNK

_T1_VB = 65536
_T1_GRID = (VOCAB + _T1_VB - 1) // _T1_VB
_PERM_ROWS = _T1_GRID * _T1_VB // 8
_PERM_VOCAB = _PERM_ROWS * 128 // EMBED_DIM


def _gather_body(tokens_hbm, table_hbm, out_hbm, idx_v, rows_v,
                 gsem0, gsem1, wsem0, wsem1):
    wid = lax.axis_index("s") * NC + lax.axis_index("c")
    base = wid * PER_W
    gsems = (gsem0, gsem1)
    wsems = (wsem0, wsem1)

    pltpu.sync_copy(tokens_hbm.at[pl.ds(base, PER_W)], idx_v)

    def xform(j, _):
        v = idx_v[pl.ds(j * 16, 16)]
        rho = ((v & -1024) + ((v & 127) << 3)
               + ((v >> 7) & 7))
        idx_v[pl.ds(j * 16, 16)] = rho
        return 0

    lax.fori_loop(0, PER_W // 16, xform, 0, unroll=8)

    def start_gather(i, b):
        return pltpu.async_copy(
            table_hbm.at[idx_v.at[pl.ds(i * CHUNK, CHUNK)]],
            rows_v.at[b], gsems[b])

    def start_write(i, b):
        return pltpu.async_copy(
            rows_v.at[b], out_hbm.at[pl.ds(base + i * CHUNK, CHUNK)],
            wsems[b])

    g = [None, None]
    w = [None, None]
    g[0] = start_gather(0, 0)
    for i in range(N_CHUNKS):
        b = i & 1
        if i + 1 < N_CHUNKS:
            if w[1 - b] is not None:
                w[1 - b].wait()
            g[1 - b] = start_gather(i + 1, 1 - b)
        g[b].wait()
        w[b] = start_write(i, b)
    w[0].wait()
    w[1].wait()


_gather = functools.partial(
    pl.kernel,
    out_type=jax.ShapeDtypeStruct((TOTAL, EMBED_DIM), jnp.float32),
    mesh=plsc.VectorSubcoreMesh(core_axis_name="c", subcore_axis_name="s"),
    scratch_types=[
        pltpu.VMEM((PER_W,), jnp.int32),
        pltpu.VMEM((2, CHUNK, EMBED_DIM), jnp.float32),
        pltpu.SemaphoreType.DMA,
        pltpu.SemaphoreType.DMA,
        pltpu.SemaphoreType.DMA,
        pltpu.SemaphoreType.DMA,
    ],
    compiler_params=pltpu.CompilerParams(use_tc_tiling_on_sc=False),
)(_gather_body)


def _t1_body(tt_ref, out_ref):
    for k in range(_T1_VB // 1024):
        base = 1024 * k
        m = jnp.concatenate(
            [tt_ref[:, base + 128 * c:base + 128 * (c + 1)]
             for c in range(8)], axis=0)
        out_ref[128 * k:128 * (k + 1), :] = m.T


_table_to_rowmajor = pl.pallas_call(
    _t1_body,
    grid=(_T1_GRID,),
    in_specs=[pl.BlockSpec((16, _T1_VB), lambda i: (0, i))],
    out_specs=pl.BlockSpec((_T1_VB // 8, 128), lambda i: (i, 0)),
    out_shape=jax.ShapeDtypeStruct((_PERM_ROWS, 128), jnp.float32),
)

_T2_BB = 512


def _t2_body(rows_ref, out_ref):
    x = rows_ref[...]
    out_ref[...] = x.T.reshape(SEQ_LEN, EMBED_DIM, _T2_BB)


_rows_to_features = pl.pallas_call(
    _t2_body,
    grid=(BATCH // _T2_BB,),
    in_specs=[pl.BlockSpec((_T2_BB, SEQ_LEN * EMBED_DIM), lambda i: (i, 0))],
    out_specs=pl.BlockSpec((SEQ_LEN, EMBED_DIM, _T2_BB), lambda i: (0, 0, i)),
    out_shape=jax.ShapeDtypeStruct((SEQ_LEN, EMBED_DIM, BATCH), jnp.float32),
)


def _mask_body(tokens_ref, pad_ref, seq_ref):
    pad_ref[...] = tokens_ref[...] == PADDING_IDX
    row = lax.broadcasted_iota(jnp.int32, (SEQ_LEN, SEQ_LEN), 0)
    col = lax.broadcasted_iota(jnp.int32, (SEQ_LEN, SEQ_LEN), 1)
    seq_ref[...] = col > row


_masks = pl.pallas_call(
    _mask_body,
    out_shape=(
        jax.ShapeDtypeStruct((BATCH, SEQ_LEN), jnp.bool_),
        jax.ShapeDtypeStruct((SEQ_LEN, SEQ_LEN), jnp.bool_),
    ),
)


def kernel(tokens, table):
    flat = tokens.reshape(TOTAL)
    table_lin = (_table_to_rowmajor(table.T)
                 .reshape(_PERM_ROWS * 128)
                 .reshape(_PERM_VOCAB, EMBED_DIM))
    rows = _gather(flat, table_lin)
    feat_t = _rows_to_features(
        rows.reshape(TOTAL * EMBED_DIM)
            .reshape(BATCH, SEQ_LEN * EMBED_DIM))
    features = feat_t.transpose(2, 0, 1)
    pad, seqm = _masks(tokens)
    return (features, (pad, seqm))
